# baseline (speedup 1.0000x reference)
import jax
import jax.numpy as jnp
from jax import lax
from jax.experimental import pallas as pl
from jax.experimental.pallas import tpu as pltpu
from jax.experimental.pallas import tpu_sc as plsc

N_ATOMS = 100000
N_PAIRS = 3200000
N_FEAT = 128
N_BASIS = 8
N_ROWS = 95
CUTOFF = 5.0
MAX_NORM = float(N_FEAT)

NW = 32

A_CHUNK = 128
A_NCHUNK = 782
A_PAD = A_NCHUNK * A_CHUNK
A_FULL = N_ATOMS // A_CHUNK
A_REM = N_ATOMS - A_FULL * A_CHUNK

P_CHUNK = 2560
P_NCHUNK = N_PAIRS // P_CHUNK
P_ITER = P_CHUNK // 16


def _ceil_div(a, b):
    return (a + b - 1) // b


def _norm_body(af_ref, out_ref):
    af = af_ref[:]
    ss = jnp.sum(af * af, axis=1, keepdims=True)
    norm = jnp.sqrt(ss + 1e-12)
    scale = jnp.minimum(1.0, MAX_NORM / norm)
    rows = lax.broadcasted_iota(jnp.int32, af.shape, 0)
    out_ref[:] = jnp.where(rows == 0, 0.0, af * scale)


def _normalize_table(atom_features):
    return pl.pallas_call(
        _norm_body,
        out_shape=jax.ShapeDtypeStruct((N_ROWS, N_FEAT), jnp.float32),
    )(atom_features)


def _sc_pairs_body(ii_ref, jj_ref, xs_ref, ys_ref, zs_ref,
                   xi_out, yi_out, zi_out, xj_out, yj_out, zj_out,
                   plane_v, ii_v, jj_v, a_v, b_v, sem):
    wid = lax.axis_index("s") * 2 + lax.axis_index("c")
    g = jnp.where(wid < 11, 0, jnp.where(wid < 22, 1, 2))
    rank = wid - jnp.where(wid < 11, 0, jnp.where(wid < 22, 11, 22))
    gsize = jnp.where(g == 2, 10, 11)

    @pl.when(g == 0)
    def _():
        pltpu.sync_copy(xs_ref, plane_v)

    @pl.when(g == 1)
    def _():
        pltpu.sync_copy(ys_ref, plane_v)

    @pl.when(g == 2)
    def _():
        pltpu.sync_copy(zs_ref, plane_v)

    def chunk_body(t, carry):
        c = rank + gsize * t

        @pl.when(c < P_NCHUNK)
        def _():
            base = c * P_CHUNK
            pltpu.sync_copy(ii_ref.at[pl.ds(base, P_CHUNK)], ii_v)
            pltpu.sync_copy(jj_ref.at[pl.ds(base, P_CHUNK)], jj_v)

            def gather16(t2, carry2):
                s = pl.ds(t2 * 16, 16)
                a_v[s] = plsc.load_gather(plane_v, [ii_v[s]])
                b_v[s] = plsc.load_gather(plane_v, [jj_v[s]])
                return carry2

            lax.fori_loop(0, P_ITER, gather16, 0, unroll=8)

            @pl.when(g == 0)
            def _():
                pltpu.sync_copy(a_v, xi_out.at[pl.ds(base, P_CHUNK)])
                pltpu.sync_copy(b_v, xj_out.at[pl.ds(base, P_CHUNK)])

            @pl.when(g == 1)
            def _():
                pltpu.sync_copy(a_v, yi_out.at[pl.ds(base, P_CHUNK)])
                pltpu.sync_copy(b_v, yj_out.at[pl.ds(base, P_CHUNK)])

            @pl.when(g == 2)
            def _():
                pltpu.sync_copy(a_v, zi_out.at[pl.ds(base, P_CHUNK)])
                pltpu.sync_copy(b_v, zj_out.at[pl.ds(base, P_CHUNK)])

        return carry

    lax.fori_loop(0, _ceil_div(P_NCHUNK, 10), chunk_body, 0)


def _sc_pairs(idx_i, idx_j, xs, ys, zs):
    mesh = plsc.VectorSubcoreMesh(core_axis_name="c", subcore_axis_name="s",
                                  num_cores=2, num_subcores=16)
    pvec = jax.ShapeDtypeStruct((N_PAIRS,), jnp.float32)
    fn = pl.kernel(
        _sc_pairs_body,
        out_type=[pvec, pvec, pvec, pvec, pvec, pvec],
        mesh=mesh,
        compiler_params=pltpu.CompilerParams(use_tc_tiling_on_sc=False,
                                             needs_layout_passes=False),
        scratch_types=[
            pltpu.VMEM((N_ATOMS,), jnp.float32),
            pltpu.VMEM((P_CHUNK,), jnp.int32),
            pltpu.VMEM((P_CHUNK,), jnp.int32),
            pltpu.VMEM((P_CHUNK,), jnp.float32),
            pltpu.VMEM((P_CHUNK,), jnp.float32),
            pltpu.SemaphoreType.DMA,
        ],
    )
    return fn(idx_i, idx_j, xs, ys, zs)


def _sc_feat_body(an_ref, table_ref, feat_out, aidx_v, frows_v, sem):
    wid = lax.axis_index("s") * 2 + lax.axis_index("c")

    def feat_chunk(t, carry):
        c = wid + NW * t

        @pl.when(c < A_NCHUNK)
        def _():
            base = c * A_CHUNK
            pltpu.sync_copy(an_ref.at[pl.ds(base, A_CHUNK)], aidx_v)
            pltpu.async_copy(table_ref.at[aidx_v], frows_v, sem).wait()

            @pl.when(c < A_FULL)
            def _():
                pltpu.sync_copy(frows_v, feat_out.at[pl.ds(base, A_CHUNK)])

            @pl.when(c == A_FULL)
            def _():
                pltpu.sync_copy(frows_v.at[pl.ds(0, A_REM)],
                                feat_out.at[pl.ds(base, A_REM)])

        return carry

    lax.fori_loop(0, _ceil_div(A_NCHUNK, NW), feat_chunk, 0)


def _sc_features(an_pad, table):
    mesh = plsc.VectorSubcoreMesh(core_axis_name="c", subcore_axis_name="s",
                                  num_cores=2, num_subcores=16)
    fn = pl.kernel(
        _sc_feat_body,
        out_type=jax.ShapeDtypeStruct((N_ATOMS, N_FEAT), jnp.float32),
        mesh=mesh,
        compiler_params=pltpu.CompilerParams(use_tc_tiling_on_sc=False),
        scratch_types=[
            pltpu.VMEM((A_CHUNK,), jnp.int32),
            pltpu.VMEM((A_CHUNK, N_FEAT), jnp.float32),
            pltpu.SemaphoreType.DMA,
        ],
    )
    return fn(an_pad, table)


R_BLK = 200
R_ROWS = N_PAIRS // 128


def _rbf_body(xi_ref, yi_ref, zi_ref, xj_ref, yj_ref, zj_ref, f3_ref,
              out_ref):
    dx = xj_ref[:] - xi_ref[:]
    dy = yj_ref[:] - yi_ref[:]
    dz = zj_ref[:] - zi_ref[:]
    d2 = dx * dx + dy * dy + dz * dz
    d = jnp.sqrt(d2 + 1e-12)
    x = d * (1.0 / CUTOFF)
    x3 = x * x * x
    fc = 1.0 + x3 * (-10.0 + x * (15.0 - 6.0 * x))
    fc = jnp.where(d < CUTOFF, fc, 0.0)
    r = d.shape[0]
    d3 = jnp.broadcast_to(d[:, None, :], (r, N_BASIS, 128))
    fc3 = jnp.broadcast_to(fc[:, None, :], (r, N_BASIS, 128))
    u = d3 * f3_ref[:]
    u = u - jnp.round(u)
    u2 = u * u
    s = u * (6.2830884630 + u2 * (-41.333247542 + u2 * (
        81.400089767 + u2 * (-74.675883870 + u2 * 33.168094613))))
    out_ref[:] = s * fc3


def _rbf_stage(xi, yi, zi, xj, yj, zj, f3):
    plane = pl.BlockSpec((R_BLK, 128), lambda i: (i, 0))
    return pl.pallas_call(
        _rbf_body,
        grid=(R_ROWS // R_BLK,),
        in_specs=[plane, plane, plane, plane, plane, plane,
                  pl.BlockSpec((1, N_BASIS, 128), lambda i: (0, 0, 0))],
        out_specs=pl.BlockSpec((R_BLK, N_BASIS, 128), lambda i: (i, 0, 0)),
        out_shape=jax.ShapeDtypeStruct((R_ROWS, N_BASIS, 128), jnp.float32),
    )(xi, yi, zi, xj, yj, zj, f3)


def kernel(atomic_numbers, positions, idx_i, idx_j, atom_features, rbf_freqs):
    table = _normalize_table(atom_features)
    an_pad = jnp.concatenate(
        [atomic_numbers.astype(jnp.int32),
         jnp.zeros((A_PAD - N_ATOMS,), jnp.int32)])
    xs = positions[:, 0]
    ys = positions[:, 1]
    zs = positions[:, 2]
    xi, yi, zi, xj, yj, zj = _sc_pairs(
        idx_i.astype(jnp.int32), idx_j.astype(jnp.int32), xs, ys, zs)
    features = _sc_features(an_pad, table)
    f3 = jnp.broadcast_to(
        (rbf_freqs * (1.0 / (2.0 * jnp.pi)))[None, :, None],
        (1, N_BASIS, 128))
    rbf8 = _rbf_stage(xi.reshape(R_ROWS, 128), yi.reshape(R_ROWS, 128),
                      zi.reshape(R_ROWS, 128), xj.reshape(R_ROWS, 128),
                      yj.reshape(R_ROWS, 128), zj.reshape(R_ROWS, 128),
                      f3)
    rbfs = rbf8.transpose(0, 2, 1).reshape(N_PAIRS, N_BASIS)
    return features, rbfs


# speedup vs baseline: 1.5521x; 1.5521x over previous
import jax
import jax.numpy as jnp
from jax import lax
from jax.experimental import pallas as pl
from jax.experimental.pallas import tpu as pltpu
from jax.experimental.pallas import tpu_sc as plsc

N_ATOMS = 100000
N_PAIRS = 3200000
N_FEAT = 128
N_BASIS = 8
N_ROWS = 95
CUTOFF = 5.0
MAX_NORM = float(N_FEAT)

NW = 32

A_CHUNK = 128
A_NCHUNK = 782
A_PAD = A_NCHUNK * A_CHUNK
A_FULL = N_ATOMS // A_CHUNK
A_REM = N_ATOMS - A_FULL * A_CHUNK

P_CHUNK = 2560
P_NCHUNK = N_PAIRS // P_CHUNK
P_SUB = P_CHUNK // 128


def _ceil_div(a, b):
    return (a + b - 1) // b


def _norm_body(af_ref, out_ref):
    af = af_ref[:]
    ss = jnp.sum(af * af, axis=1, keepdims=True)
    norm = jnp.sqrt(ss + 1e-12)
    scale = jnp.minimum(1.0, MAX_NORM / norm)
    rows = lax.broadcasted_iota(jnp.int32, af.shape, 0)
    out_ref[:] = jnp.where(rows == 0, 0.0, af * scale)


def _normalize_table(atom_features):
    return pl.pallas_call(
        _norm_body,
        out_shape=jax.ShapeDtypeStruct((N_ROWS, N_FEAT), jnp.float32),
    )(atom_features)


def _sc_pairs_body(ii_ref, jj_ref, xs_ref, ys_ref, zs_ref,
                   xi_out, yi_out, zi_out, xj_out, yj_out, zj_out,
                   ii_v, jj_v,
                   xi_v, yi_v, zi_v, xj_v, yj_v, zj_v,
                   xs_sh, ys_sh, zs_sh, sem):
    wid = lax.axis_index("s") * 2 + lax.axis_index("c")

    @pl.when(lax.axis_index("s") == 0)
    def _():
        pltpu.sync_copy(xs_ref, xs_sh)
        pltpu.sync_copy(ys_ref, ys_sh)
        pltpu.sync_copy(zs_ref, zs_sh)

    plsc.subcore_barrier()

    def pair_chunk(t, carry):
        c = wid + NW * t

        @pl.when(c < P_NCHUNK)
        def _():
            base = c * P_CHUNK
            pltpu.sync_copy(ii_ref.at[pl.ds(base, P_CHUNK)], ii_v)
            pltpu.sync_copy(jj_ref.at[pl.ds(base, P_CHUNK)], jj_v)
            descs = []
            for k in range(P_SUB):
                s = pl.ds(k * 128, 128)
                for src, idx, dst in (
                        (xs_sh, ii_v, xi_v), (ys_sh, ii_v, yi_v),
                        (zs_sh, ii_v, zi_v), (xs_sh, jj_v, xj_v),
                        (ys_sh, jj_v, yj_v), (zs_sh, jj_v, zj_v)):
                    descs.append(pltpu.async_copy(
                        src.at[idx.at[s]], dst.at[s], sem))
            for d in descs:
                d.wait()
            for buf, out in ((xi_v, xi_out), (yi_v, yi_out), (zi_v, zi_out),
                             (xj_v, xj_out), (yj_v, yj_out), (zj_v, zj_out)):
                pltpu.sync_copy(buf, out.at[pl.ds(base, P_CHUNK)])

        return carry

    lax.fori_loop(0, _ceil_div(P_NCHUNK, NW), pair_chunk, 0)


def _sc_pairs(idx_i, idx_j, xs, ys, zs):
    mesh = plsc.VectorSubcoreMesh(core_axis_name="c", subcore_axis_name="s",
                                  num_cores=2, num_subcores=16)
    pvec = jax.ShapeDtypeStruct((N_PAIRS,), jnp.float32)
    fn = pl.kernel(
        _sc_pairs_body,
        out_type=[pvec, pvec, pvec, pvec, pvec, pvec],
        mesh=mesh,
        compiler_params=pltpu.CompilerParams(use_tc_tiling_on_sc=False),
        scratch_types=[
            pltpu.VMEM((P_CHUNK,), jnp.int32),
            pltpu.VMEM((P_CHUNK,), jnp.int32),
            pltpu.VMEM((P_CHUNK,), jnp.float32),
            pltpu.VMEM((P_CHUNK,), jnp.float32),
            pltpu.VMEM((P_CHUNK,), jnp.float32),
            pltpu.VMEM((P_CHUNK,), jnp.float32),
            pltpu.VMEM((P_CHUNK,), jnp.float32),
            pltpu.VMEM((P_CHUNK,), jnp.float32),
            pltpu.VMEM_SHARED((N_ATOMS,), jnp.float32),
            pltpu.VMEM_SHARED((N_ATOMS,), jnp.float32),
            pltpu.VMEM_SHARED((N_ATOMS,), jnp.float32),
            pltpu.SemaphoreType.DMA,
        ],
    )
    return fn(idx_i, idx_j, xs, ys, zs)


def _sc_feat_body(an_ref, table_ref, feat_out, aidx_v, frows_v, sem):
    wid = lax.axis_index("s") * 2 + lax.axis_index("c")

    def feat_chunk(t, carry):
        c = wid + NW * t

        @pl.when(c < A_NCHUNK)
        def _():
            base = c * A_CHUNK
            pltpu.sync_copy(an_ref.at[pl.ds(base, A_CHUNK)], aidx_v)
            pltpu.async_copy(table_ref.at[aidx_v], frows_v, sem).wait()

            @pl.when(c < A_FULL)
            def _():
                pltpu.sync_copy(frows_v, feat_out.at[pl.ds(base, A_CHUNK)])

            @pl.when(c == A_FULL)
            def _():
                pltpu.sync_copy(frows_v.at[pl.ds(0, A_REM)],
                                feat_out.at[pl.ds(base, A_REM)])

        return carry

    lax.fori_loop(0, _ceil_div(A_NCHUNK, NW), feat_chunk, 0)


def _sc_features(an_pad, table):
    mesh = plsc.VectorSubcoreMesh(core_axis_name="c", subcore_axis_name="s",
                                  num_cores=2, num_subcores=16)
    fn = pl.kernel(
        _sc_feat_body,
        out_type=jax.ShapeDtypeStruct((N_ATOMS, N_FEAT), jnp.float32),
        mesh=mesh,
        compiler_params=pltpu.CompilerParams(use_tc_tiling_on_sc=False),
        scratch_types=[
            pltpu.VMEM((A_CHUNK,), jnp.int32),
            pltpu.VMEM((A_CHUNK, N_FEAT), jnp.float32),
            pltpu.SemaphoreType.DMA,
        ],
    )
    return fn(an_pad, table)


R_BLK = 200
R_ROWS = N_PAIRS // 128


def _rbf_body(xi_ref, yi_ref, zi_ref, xj_ref, yj_ref, zj_ref, f3_ref,
              out_ref):
    dx = xj_ref[:] - xi_ref[:]
    dy = yj_ref[:] - yi_ref[:]
    dz = zj_ref[:] - zi_ref[:]
    d2 = dx * dx + dy * dy + dz * dz
    d = jnp.sqrt(d2 + 1e-12)
    x = d * (1.0 / CUTOFF)
    x3 = x * x * x
    fc = 1.0 + x3 * (-10.0 + x * (15.0 - 6.0 * x))
    fc = jnp.where(d < CUTOFF, fc, 0.0)
    r = d.shape[0]
    d3 = jnp.broadcast_to(d[:, None, :], (r, N_BASIS, 128))
    fc3 = jnp.broadcast_to(fc[:, None, :], (r, N_BASIS, 128))
    u = d3 * f3_ref[:]
    u = u - jnp.round(u)
    u2 = u * u
    s = u * (6.2830884630 + u2 * (-41.333247542 + u2 * (
        81.400089767 + u2 * (-74.675883870 + u2 * 33.168094613))))
    out_ref[:] = s * fc3


def _rbf_stage(xi, yi, zi, xj, yj, zj, f3):
    plane = pl.BlockSpec((R_BLK, 128), lambda i: (i, 0))
    return pl.pallas_call(
        _rbf_body,
        grid=(R_ROWS // R_BLK,),
        in_specs=[plane, plane, plane, plane, plane, plane,
                  pl.BlockSpec((1, N_BASIS, 128), lambda i: (0, 0, 0))],
        out_specs=pl.BlockSpec((R_BLK, N_BASIS, 128), lambda i: (i, 0, 0)),
        out_shape=jax.ShapeDtypeStruct((R_ROWS, N_BASIS, 128), jnp.float32),
    )(xi, yi, zi, xj, yj, zj, f3)


def kernel(atomic_numbers, positions, idx_i, idx_j, atom_features, rbf_freqs):
    table = _normalize_table(atom_features)
    an_pad = jnp.concatenate(
        [atomic_numbers.astype(jnp.int32),
         jnp.zeros((A_PAD - N_ATOMS,), jnp.int32)])
    xs = positions[:, 0]
    ys = positions[:, 1]
    zs = positions[:, 2]
    xi, yi, zi, xj, yj, zj = _sc_pairs(
        idx_i.astype(jnp.int32), idx_j.astype(jnp.int32), xs, ys, zs)
    features = _sc_features(an_pad, table)
    f3 = jnp.broadcast_to(
        (rbf_freqs * (1.0 / (2.0 * jnp.pi)))[None, :, None],
        (1, N_BASIS, 128))
    rbf8 = _rbf_stage(xi.reshape(R_ROWS, 128), yi.reshape(R_ROWS, 128),
                      zi.reshape(R_ROWS, 128), xj.reshape(R_ROWS, 128),
                      yj.reshape(R_ROWS, 128), zj.reshape(R_ROWS, 128),
                      f3)
    rbfs = rbf8.transpose(0, 2, 1).reshape(N_PAIRS, N_BASIS)
    return features, rbfs


# speedup vs baseline: 1.6515x; 1.0641x over previous
import jax
import jax.numpy as jnp
from jax import lax
from jax.experimental import pallas as pl
from jax.experimental.pallas import tpu as pltpu
from jax.experimental.pallas import tpu_sc as plsc

N_ATOMS = 100000
N_PAIRS = 3200000
N_FEAT = 128
N_BASIS = 8
N_ROWS = 95
CUTOFF = 5.0
MAX_NORM = float(N_FEAT)

NW = 32

A_CHUNK = 128
A_NCHUNK = 782
A_PAD = A_NCHUNK * A_CHUNK
A_FULL = N_ATOMS // A_CHUNK
A_REM = N_ATOMS - A_FULL * A_CHUNK

P_CHUNK = 5120
P_NCHUNK = N_PAIRS // P_CHUNK
P_SUB = P_CHUNK // 128


def _ceil_div(a, b):
    return (a + b - 1) // b


def _norm_body(af_ref, out_ref):
    af = af_ref[:]
    ss = jnp.sum(af * af, axis=1, keepdims=True)
    norm = jnp.sqrt(ss + 1e-12)
    scale = jnp.minimum(1.0, MAX_NORM / norm)
    rows = lax.broadcasted_iota(jnp.int32, af.shape, 0)
    out_ref[:] = jnp.where(rows == 0, 0.0, af * scale)


def _normalize_table(atom_features):
    return pl.pallas_call(
        _norm_body,
        out_shape=jax.ShapeDtypeStruct((N_ROWS, N_FEAT), jnp.float32),
    )(atom_features)


def _sc_pairs_body(ii_ref, jj_ref, xs_ref, ys_ref, zs_ref,
                   xi_out, yi_out, zi_out, xj_out, yj_out, zj_out,
                   ii_v, jj_v,
                   xi_v, yi_v, zi_v, xj_v, yj_v, zj_v,
                   xs_sh, ys_sh, zs_sh, sem):
    wid = lax.axis_index("s") * 2 + lax.axis_index("c")

    @pl.when(lax.axis_index("s") == 0)
    def _():
        pltpu.sync_copy(xs_ref, xs_sh)
        pltpu.sync_copy(ys_ref, ys_sh)
        pltpu.sync_copy(zs_ref, zs_sh)

    plsc.subcore_barrier()

    def pair_chunk(t, carry):
        c = wid + NW * t

        @pl.when(c < P_NCHUNK)
        def _():
            base = c * P_CHUNK
            pltpu.sync_copy(ii_ref.at[pl.ds(base, P_CHUNK)], ii_v)
            pltpu.sync_copy(jj_ref.at[pl.ds(base, P_CHUNK)], jj_v)
            descs = []
            for k in range(P_SUB):
                s = pl.ds(k * 128, 128)
                for src, idx, dst in (
                        (xs_sh, ii_v, xi_v), (ys_sh, ii_v, yi_v),
                        (zs_sh, ii_v, zi_v), (xs_sh, jj_v, xj_v),
                        (ys_sh, jj_v, yj_v), (zs_sh, jj_v, zj_v)):
                    descs.append(pltpu.async_copy(
                        src.at[idx.at[s]], dst.at[s], sem))
            for d in descs:
                d.wait()
            for buf, out in ((xi_v, xi_out), (yi_v, yi_out), (zi_v, zi_out),
                             (xj_v, xj_out), (yj_v, yj_out), (zj_v, zj_out)):
                pltpu.sync_copy(buf, out.at[pl.ds(base, P_CHUNK)])

        return carry

    lax.fori_loop(0, _ceil_div(P_NCHUNK, NW), pair_chunk, 0)


def _sc_pairs(idx_i, idx_j, xs, ys, zs):
    mesh = plsc.VectorSubcoreMesh(core_axis_name="c", subcore_axis_name="s",
                                  num_cores=2, num_subcores=16)
    pvec = jax.ShapeDtypeStruct((N_PAIRS,), jnp.float32)
    fn = pl.kernel(
        _sc_pairs_body,
        out_type=[pvec, pvec, pvec, pvec, pvec, pvec],
        mesh=mesh,
        compiler_params=pltpu.CompilerParams(use_tc_tiling_on_sc=False),
        scratch_types=[
            pltpu.VMEM((P_CHUNK,), jnp.int32),
            pltpu.VMEM((P_CHUNK,), jnp.int32),
            pltpu.VMEM((P_CHUNK,), jnp.float32),
            pltpu.VMEM((P_CHUNK,), jnp.float32),
            pltpu.VMEM((P_CHUNK,), jnp.float32),
            pltpu.VMEM((P_CHUNK,), jnp.float32),
            pltpu.VMEM((P_CHUNK,), jnp.float32),
            pltpu.VMEM((P_CHUNK,), jnp.float32),
            pltpu.VMEM_SHARED((N_ATOMS,), jnp.float32),
            pltpu.VMEM_SHARED((N_ATOMS,), jnp.float32),
            pltpu.VMEM_SHARED((N_ATOMS,), jnp.float32),
            pltpu.SemaphoreType.DMA,
        ],
    )
    return fn(idx_i, idx_j, xs, ys, zs)


def _sc_feat_body(an_ref, table_ref, feat_out, aidx_v, frows_v, sem):
    wid = lax.axis_index("s") * 2 + lax.axis_index("c")

    def feat_chunk(t, carry):
        c = wid + NW * t

        @pl.when(c < A_NCHUNK)
        def _():
            base = c * A_CHUNK
            pltpu.sync_copy(an_ref.at[pl.ds(base, A_CHUNK)], aidx_v)
            pltpu.async_copy(table_ref.at[aidx_v], frows_v, sem).wait()

            @pl.when(c < A_FULL)
            def _():
                pltpu.sync_copy(frows_v, feat_out.at[pl.ds(base, A_CHUNK)])

            @pl.when(c == A_FULL)
            def _():
                pltpu.sync_copy(frows_v.at[pl.ds(0, A_REM)],
                                feat_out.at[pl.ds(base, A_REM)])

        return carry

    lax.fori_loop(0, _ceil_div(A_NCHUNK, NW), feat_chunk, 0)


def _sc_features(an_pad, table):
    mesh = plsc.VectorSubcoreMesh(core_axis_name="c", subcore_axis_name="s",
                                  num_cores=2, num_subcores=16)
    fn = pl.kernel(
        _sc_feat_body,
        out_type=jax.ShapeDtypeStruct((N_ATOMS, N_FEAT), jnp.float32),
        mesh=mesh,
        compiler_params=pltpu.CompilerParams(use_tc_tiling_on_sc=False),
        scratch_types=[
            pltpu.VMEM((A_CHUNK,), jnp.int32),
            pltpu.VMEM((A_CHUNK, N_FEAT), jnp.float32),
            pltpu.SemaphoreType.DMA,
        ],
    )
    return fn(an_pad, table)


R_BLK = 200
R_ROWS = N_PAIRS // 128


def _rbf_body(xi_ref, yi_ref, zi_ref, xj_ref, yj_ref, zj_ref, f3_ref,
              out_ref):
    dx = xj_ref[:] - xi_ref[:]
    dy = yj_ref[:] - yi_ref[:]
    dz = zj_ref[:] - zi_ref[:]
    d2 = dx * dx + dy * dy + dz * dz
    d = jnp.sqrt(d2 + 1e-12)
    x = d * (1.0 / CUTOFF)
    x3 = x * x * x
    fc = 1.0 + x3 * (-10.0 + x * (15.0 - 6.0 * x))
    fc = jnp.where(d < CUTOFF, fc, 0.0)
    r = d.shape[0]
    d3 = jnp.broadcast_to(d[:, None, :], (r, N_BASIS, 128))
    fc3 = jnp.broadcast_to(fc[:, None, :], (r, N_BASIS, 128))
    u = d3 * f3_ref[:]
    u = u - jnp.round(u)
    u2 = u * u
    s = u * (6.2830884630 + u2 * (-41.333247542 + u2 * (
        81.400089767 + u2 * (-74.675883870 + u2 * 33.168094613))))
    out_ref[:] = s * fc3


def _rbf_stage(xi, yi, zi, xj, yj, zj, f3):
    plane = pl.BlockSpec((R_BLK, 128), lambda i: (i, 0))
    return pl.pallas_call(
        _rbf_body,
        grid=(R_ROWS // R_BLK,),
        in_specs=[plane, plane, plane, plane, plane, plane,
                  pl.BlockSpec((1, N_BASIS, 128), lambda i: (0, 0, 0))],
        out_specs=pl.BlockSpec((R_BLK, N_BASIS, 128), lambda i: (i, 0, 0)),
        out_shape=jax.ShapeDtypeStruct((R_ROWS, N_BASIS, 128), jnp.float32),
    )(xi, yi, zi, xj, yj, zj, f3)


def kernel(atomic_numbers, positions, idx_i, idx_j, atom_features, rbf_freqs):
    table = _normalize_table(atom_features)
    an_pad = jnp.concatenate(
        [atomic_numbers.astype(jnp.int32),
         jnp.zeros((A_PAD - N_ATOMS,), jnp.int32)])
    xs = positions[:, 0]
    ys = positions[:, 1]
    zs = positions[:, 2]
    xi, yi, zi, xj, yj, zj = _sc_pairs(
        idx_i.astype(jnp.int32), idx_j.astype(jnp.int32), xs, ys, zs)
    features = _sc_features(an_pad, table)
    f3 = jnp.broadcast_to(
        (rbf_freqs * (1.0 / (2.0 * jnp.pi)))[None, :, None],
        (1, N_BASIS, 128))
    rbf8 = _rbf_stage(xi.reshape(R_ROWS, 128), yi.reshape(R_ROWS, 128),
                      zi.reshape(R_ROWS, 128), xj.reshape(R_ROWS, 128),
                      yj.reshape(R_ROWS, 128), zj.reshape(R_ROWS, 128),
                      f3)
    rbfs = rbf8.transpose(0, 2, 1).reshape(N_PAIRS, N_BASIS)
    return features, rbfs


# speedup vs baseline: 1.6718x; 1.0123x over previous
import jax
import jax.numpy as jnp
from jax import lax
from jax.experimental import pallas as pl
from jax.experimental.pallas import tpu as pltpu
from jax.experimental.pallas import tpu_sc as plsc

N_ATOMS = 100000
N_PAIRS = 3200000
N_FEAT = 128
N_BASIS = 8
N_ROWS = 95
CUTOFF = 5.0
MAX_NORM = float(N_FEAT)

NW = 32

A_CHUNK = 128
A_NCHUNK = 782
A_PAD = A_NCHUNK * A_CHUNK
A_FULL = N_ATOMS // A_CHUNK
A_REM = N_ATOMS - A_FULL * A_CHUNK

P_CHUNK = 6400
P_NCHUNK = N_PAIRS // P_CHUNK
P_SUB = P_CHUNK // 128


def _ceil_div(a, b):
    return (a + b - 1) // b


def _norm_body(af_ref, out_ref):
    af = af_ref[:]
    ss = jnp.sum(af * af, axis=1, keepdims=True)
    norm = jnp.sqrt(ss + 1e-12)
    scale = jnp.minimum(1.0, MAX_NORM / norm)
    rows = lax.broadcasted_iota(jnp.int32, af.shape, 0)
    out_ref[:] = jnp.where(rows == 0, 0.0, af * scale)


def _normalize_table(atom_features):
    return pl.pallas_call(
        _norm_body,
        out_shape=jax.ShapeDtypeStruct((N_ROWS, N_FEAT), jnp.float32),
    )(atom_features)


def _sc_pairs_body(ii_ref, jj_ref, xs_ref, ys_ref, zs_ref,
                   xi_out, yi_out, zi_out, xj_out, yj_out, zj_out,
                   ii_v, jj_v,
                   xi_v, yi_v, zi_v, xj_v, yj_v, zj_v,
                   xs_sh, ys_sh, zs_sh, sem):
    wid = lax.axis_index("s") * 2 + lax.axis_index("c")

    @pl.when(lax.axis_index("s") == 0)
    def _():
        pltpu.sync_copy(xs_ref, xs_sh)
        pltpu.sync_copy(ys_ref, ys_sh)
        pltpu.sync_copy(zs_ref, zs_sh)

    plsc.subcore_barrier()

    def pair_chunk(t, carry):
        c = wid + NW * t

        @pl.when(c < P_NCHUNK)
        def _():
            base = c * P_CHUNK
            pltpu.sync_copy(ii_ref.at[pl.ds(base, P_CHUNK)], ii_v)
            pltpu.sync_copy(jj_ref.at[pl.ds(base, P_CHUNK)], jj_v)
            descs = []
            for k in range(P_SUB):
                s = pl.ds(k * 128, 128)
                for src, idx, dst in (
                        (xs_sh, ii_v, xi_v), (ys_sh, ii_v, yi_v),
                        (zs_sh, ii_v, zi_v), (xs_sh, jj_v, xj_v),
                        (ys_sh, jj_v, yj_v), (zs_sh, jj_v, zj_v)):
                    descs.append(pltpu.async_copy(
                        src.at[idx.at[s]], dst.at[s], sem))
            for d in descs:
                d.wait()
            for buf, out in ((xi_v, xi_out), (yi_v, yi_out), (zi_v, zi_out),
                             (xj_v, xj_out), (yj_v, yj_out), (zj_v, zj_out)):
                pltpu.sync_copy(buf, out.at[pl.ds(base, P_CHUNK)])

        return carry

    lax.fori_loop(0, _ceil_div(P_NCHUNK, NW), pair_chunk, 0)


def _sc_pairs(idx_i, idx_j, xs, ys, zs):
    mesh = plsc.VectorSubcoreMesh(core_axis_name="c", subcore_axis_name="s",
                                  num_cores=2, num_subcores=16)
    pvec = jax.ShapeDtypeStruct((N_PAIRS,), jnp.float32)
    fn = pl.kernel(
        _sc_pairs_body,
        out_type=[pvec, pvec, pvec, pvec, pvec, pvec],
        mesh=mesh,
        compiler_params=pltpu.CompilerParams(use_tc_tiling_on_sc=False),
        scratch_types=[
            pltpu.VMEM((P_CHUNK,), jnp.int32),
            pltpu.VMEM((P_CHUNK,), jnp.int32),
            pltpu.VMEM((P_CHUNK,), jnp.float32),
            pltpu.VMEM((P_CHUNK,), jnp.float32),
            pltpu.VMEM((P_CHUNK,), jnp.float32),
            pltpu.VMEM((P_CHUNK,), jnp.float32),
            pltpu.VMEM((P_CHUNK,), jnp.float32),
            pltpu.VMEM((P_CHUNK,), jnp.float32),
            pltpu.VMEM_SHARED((N_ATOMS,), jnp.float32),
            pltpu.VMEM_SHARED((N_ATOMS,), jnp.float32),
            pltpu.VMEM_SHARED((N_ATOMS,), jnp.float32),
            pltpu.SemaphoreType.DMA,
        ],
    )
    return fn(idx_i, idx_j, xs, ys, zs)


def _sc_feat_body(an_ref, table_ref, feat_out, aidx_v, frows_v, sem):
    wid = lax.axis_index("s") * 2 + lax.axis_index("c")

    def feat_chunk(t, carry):
        c = wid + NW * t

        @pl.when(c < A_NCHUNK)
        def _():
            base = c * A_CHUNK
            pltpu.sync_copy(an_ref.at[pl.ds(base, A_CHUNK)], aidx_v)
            pltpu.async_copy(table_ref.at[aidx_v], frows_v, sem).wait()

            @pl.when(c < A_FULL)
            def _():
                pltpu.sync_copy(frows_v, feat_out.at[pl.ds(base, A_CHUNK)])

            @pl.when(c == A_FULL)
            def _():
                pltpu.sync_copy(frows_v.at[pl.ds(0, A_REM)],
                                feat_out.at[pl.ds(base, A_REM)])

        return carry

    lax.fori_loop(0, _ceil_div(A_NCHUNK, NW), feat_chunk, 0)


def _sc_features(an_pad, table):
    mesh = plsc.VectorSubcoreMesh(core_axis_name="c", subcore_axis_name="s",
                                  num_cores=2, num_subcores=16)
    fn = pl.kernel(
        _sc_feat_body,
        out_type=jax.ShapeDtypeStruct((N_ATOMS, N_FEAT), jnp.float32),
        mesh=mesh,
        compiler_params=pltpu.CompilerParams(use_tc_tiling_on_sc=False),
        scratch_types=[
            pltpu.VMEM((A_CHUNK,), jnp.int32),
            pltpu.VMEM((A_CHUNK, N_FEAT), jnp.float32),
            pltpu.SemaphoreType.DMA,
        ],
    )
    return fn(an_pad, table)


R_BLK = 200
R_ROWS = N_PAIRS // 128


def _rbf_body(xi_ref, yi_ref, zi_ref, xj_ref, yj_ref, zj_ref, f3_ref,
              out_ref):
    dx = xj_ref[:] - xi_ref[:]
    dy = yj_ref[:] - yi_ref[:]
    dz = zj_ref[:] - zi_ref[:]
    d2 = dx * dx + dy * dy + dz * dz
    d = jnp.sqrt(d2 + 1e-12)
    x = d * (1.0 / CUTOFF)
    x3 = x * x * x
    fc = 1.0 + x3 * (-10.0 + x * (15.0 - 6.0 * x))
    fc = jnp.where(d < CUTOFF, fc, 0.0)
    r = d.shape[0]
    d3 = jnp.broadcast_to(d[:, None, :], (r, N_BASIS, 128))
    fc3 = jnp.broadcast_to(fc[:, None, :], (r, N_BASIS, 128))
    u = d3 * f3_ref[:]
    u = u - jnp.round(u)
    u2 = u * u
    s = u * (6.2830884630 + u2 * (-41.333247542 + u2 * (
        81.400089767 + u2 * (-74.675883870 + u2 * 33.168094613))))
    out_ref[:] = s * fc3


def _rbf_stage(xi, yi, zi, xj, yj, zj, f3):
    plane = pl.BlockSpec((R_BLK, 128), lambda i: (i, 0))
    return pl.pallas_call(
        _rbf_body,
        grid=(R_ROWS // R_BLK,),
        in_specs=[plane, plane, plane, plane, plane, plane,
                  pl.BlockSpec((1, N_BASIS, 128), lambda i: (0, 0, 0))],
        out_specs=pl.BlockSpec((R_BLK, N_BASIS, 128), lambda i: (i, 0, 0)),
        out_shape=jax.ShapeDtypeStruct((R_ROWS, N_BASIS, 128), jnp.float32),
    )(xi, yi, zi, xj, yj, zj, f3)


def kernel(atomic_numbers, positions, idx_i, idx_j, atom_features, rbf_freqs):
    table = _normalize_table(atom_features)
    an_pad = jnp.concatenate(
        [atomic_numbers.astype(jnp.int32),
         jnp.zeros((A_PAD - N_ATOMS,), jnp.int32)])
    xs = positions[:, 0]
    ys = positions[:, 1]
    zs = positions[:, 2]
    xi, yi, zi, xj, yj, zj = _sc_pairs(
        idx_i.astype(jnp.int32), idx_j.astype(jnp.int32), xs, ys, zs)
    features = _sc_features(an_pad, table)
    f3 = jnp.broadcast_to(
        (rbf_freqs * (1.0 / (2.0 * jnp.pi)))[None, :, None],
        (1, N_BASIS, 128))
    rbf8 = _rbf_stage(xi.reshape(R_ROWS, 128), yi.reshape(R_ROWS, 128),
                      zi.reshape(R_ROWS, 128), xj.reshape(R_ROWS, 128),
                      yj.reshape(R_ROWS, 128), zj.reshape(R_ROWS, 128),
                      f3)
    rbfs = rbf8.transpose(0, 2, 1).reshape(N_PAIRS, N_BASIS)
    return features, rbfs
